# baseline (speedup 1.0000x reference)
import numpy as np
import jax
import jax.numpy as jnp
from jax.experimental import pallas as pl

B = 128
S = 256
DIM = 768
EMB = 256
K = 8192
EPS = 1e-12
G = 8
NIMG = 2 * B
KT = 1024


def _np_consts():
    sel1 = np.zeros((9 * 64, 256), np.float32)
    for dh in range(3):
        for dw in range(3):
            t = dh * 3 + dw
            for i in range(8):
                for j in range(8):
                    h, w = 2 * i + dh - 1, 2 * j + dw - 1
                    if 0 <= h < 16 and 0 <= w < 16:
                        sel1[t * 64 + i * 8 + j, h * 16 + w] = 1.0
    sel2 = np.zeros((9 * 16 * G, 64 * G), np.float32)
    for dh in range(3):
        for dw in range(3):
            t = dh * 3 + dw
            for g in range(G):
                for i in range(4):
                    for j in range(4):
                        h, w = 2 * i + dh - 1, 2 * j + dw - 1
                        if 0 <= h < 8 and 0 <= w < 8:
                            sel2[t * 16 * G + g * 16 + i * 4 + j,
                                 g * 64 + h * 8 + w] = 1.0
    sel3 = np.zeros((9 * 4 * G, 16 * G), np.float32)
    for dh in range(3):
        for dw in range(3):
            t = dh * 3 + dw
            for g in range(G):
                for i in range(2):
                    for j in range(2):
                        h, w = i + dh, j + dw
                        sel3[t * 4 * G + g * 4 + i * 2 + j,
                             g * 16 + h * 4 + w] = 1.0
    gcat = np.zeros((4 * EMB, 4 * EMB), np.float32)
    for i in range(4):
        for j in range(EMB):
            p, c = j // 64, 4 * (j % 64) + i
            gcat[p * EMB + c, i * EMB + j] = 1.0
    return sel1, sel2, sel3, gcat


_SEL1, _SEL2, _SEL3, _GCAT = _np_consts()


def _encoder_body(x_ref, win_ref, bin_ref, sel1_ref, sel2_ref, sel3_ref,
                  w1_ref, b1_ref, w2_ref, b2_ref, w3_ref, b3_ref, z_ref):
    f32 = jnp.float32
    X = x_ref[...].reshape(G * S, DIM)
    Y = jnp.dot(X, win_ref[...], preferred_element_type=f32) + bin_ref[...]
    sel1 = sel1_ref[...]
    t1 = jnp.concatenate(
        [jnp.dot(sel1, Y[g * S:(g + 1) * S], preferred_element_type=f32)[None]
         for g in range(G)], axis=0)
    a1 = jnp.zeros((G * 64, EMB), f32)
    for t in range(9):
        xt = t1[:, t * 64:(t + 1) * 64, :].reshape(G * 64, EMB)
        a1 = a1 + jnp.dot(xt, w1_ref[t], preferred_element_type=f32)
    a1 = jax.nn.relu(a1 + b1_ref[...])
    t2 = jnp.dot(sel2_ref[...], a1, preferred_element_type=f32)
    a2 = jnp.zeros((G * 16, EMB), f32)
    for t in range(9):
        a2 = a2 + jnp.dot(t2[t * 16 * G:(t + 1) * 16 * G], w2_ref[t],
                          preferred_element_type=f32)
    a2 = jax.nn.relu(a2 + b2_ref[...])
    t3 = jnp.dot(sel3_ref[...], a2, preferred_element_type=f32)
    z = jnp.zeros((G * 4, EMB), f32)
    for t in range(9):
        z = z + jnp.dot(t3[t * 4 * G:(t + 1) * 4 * G], w3_ref[t],
                        preferred_element_type=f32)
    z_ref[...] = z + b3_ref[...]


def _vq_body(zf_ref, zl_ref, cbt_ref, cb_ref, noise_ref, q_ref, idx_ref):
    f32 = jnp.float32
    x = zl_ref[...] - zf_ref[...]
    xn2 = jnp.sum(x * x, axis=1, keepdims=True)
    best = jnp.full((4 * B, 1), jnp.inf, f32)
    bidx = jnp.zeros((4 * B, 1), jnp.int32)
    lane = jax.lax.broadcasted_iota(jnp.int32, (4 * B, KT), 1)
    for t in range(K // KT):
        cbt = cbt_ref[:, t * KT:(t + 1) * KT]
        cn2 = jnp.sum(cbt * cbt, axis=0, keepdims=True)
        s = xn2 - 2.0 * jnp.dot(x, cbt, preferred_element_type=f32) + cn2
        m = jnp.min(s, axis=1, keepdims=True)
        li = jnp.min(jnp.where(s <= m, lane, K), axis=1, keepdims=True) + t * KT
        upd = m < best
        bidx = jnp.where(upd, li, bidx)
        best = jnp.where(upd, m, best)
    hq = jnp.zeros((4 * B, EMB), f32)
    for t in range(K // KT):
        oh = (bidx == (lane + t * KT)).astype(f32)
        hq = hq + jnp.dot(oh, cb_ref[t * KT:(t + 1) * KT, :],
                          preferred_element_type=f32)
    r = x - hq
    nres = jnp.sqrt(jnp.sum(r * r, axis=1, keepdims=True))
    noise = noise_ref[...]
    nrand = jnp.sqrt(jnp.sum(noise * noise, axis=1, keepdims=True))
    q_ref[...] = x + (nres / nrand + EPS) * noise
    idx_ref[...] = bidx


def _out_body(qf_ref, idx_ref, gcat_ref, wout_ref, bout_ref, out_ref, ppl_ref):
    f32 = jnp.float32
    qd = jnp.dot(qf_ref[...], gcat_ref[...], preferred_element_type=f32)
    outs = []
    for i in range(4):
        o = jnp.dot(qd[:, i * EMB:(i + 1) * EMB], wout_ref[...],
                    preferred_element_type=f32) + bout_ref[...]
        outs.append(o[None])
    out_ref[...] = jnp.concatenate(outs, axis=0)
    idx = idx_ref[...]
    lane = jax.lax.broadcasted_iota(jnp.int32, (4 * B, KT), 1)
    tot = jnp.float32(0.0)
    for t in range(K // KT):
        cnt = jnp.sum((idx == (lane + t * KT)).astype(f32), axis=0,
                      keepdims=True)
        p = cnt / jnp.float32(4 * B)
        tot = tot + jnp.sum(p * jnp.log(p + 1e-10))
    ppl_ref[...] = jnp.full((1, 1), 1.0, f32) * jnp.exp(-tot)


def kernel(input_data_first, input_data_last, codebooks, W_in, b_in,
           conv1_w, conv1_b, conv2_w, conv2_b, conv3_w, conv3_b,
           W_out, b_out, noise):
    f32 = jnp.float32
    sel1 = jnp.asarray(_SEL1)
    sel2 = jnp.asarray(_SEL2)
    sel3 = jnp.asarray(_SEL3)
    gcat = jnp.asarray(_GCAT)

    inp_all = jnp.concatenate([input_data_first, input_data_last], axis=0)
    w1 = conv1_w.reshape(9, EMB, EMB)
    w2 = conv2_w.reshape(9, EMB, EMB)
    w3 = conv3_w.reshape(9, EMB, EMB)

    full = lambda shape: pl.BlockSpec(shape, lambda g: (0,) * len(shape))
    z_all = pl.pallas_call(
        _encoder_body,
        grid=(NIMG // G,),
        in_specs=[
            pl.BlockSpec((G, S, DIM), lambda g: (g, 0, 0)),
            full((DIM, EMB)),
            full((1, EMB)),
            full((9 * 64, 256)),
            full((9 * 16 * G, 64 * G)),
            full((9 * 4 * G, 16 * G)),
            full((9, EMB, EMB)),
            full((1, EMB)),
            full((9, EMB, EMB)),
            full((1, EMB)),
            full((9, EMB, EMB)),
            full((1, EMB)),
        ],
        out_specs=pl.BlockSpec((G * 4, EMB), lambda g: (g, 0)),
        out_shape=jax.ShapeDtypeStruct((NIMG * 4, EMB), f32),
    )(inp_all, W_in, b_in.reshape(1, EMB), sel1, sel2, sel3,
      w1, conv1_b.reshape(1, EMB), w2, conv2_b.reshape(1, EMB),
      w3, conv3_b.reshape(1, EMB))

    zf = z_all[:4 * B]
    zl = z_all[4 * B:]

    q, idx = pl.pallas_call(
        _vq_body,
        out_shape=(jax.ShapeDtypeStruct((4 * B, EMB), f32),
                   jax.ShapeDtypeStruct((4 * B, 1), jnp.int32)),
    )(zf, zl, codebooks.T, codebooks, noise)

    qf = q.reshape(B, 4 * EMB)
    out4, ppl = pl.pallas_call(
        _out_body,
        out_shape=(jax.ShapeDtypeStruct((4, B, DIM), f32),
                   jax.ShapeDtypeStruct((1, 1), f32)),
    )(qf, idx, gcat, W_out, b_out.reshape(1, DIM))

    out = jnp.transpose(out4, (1, 0, 2))
    return out, ppl.reshape(()), idx.reshape(4 * B)


# speedup vs baseline: 1.5273x; 1.5273x over previous
import numpy as np
import jax
import jax.numpy as jnp
from jax.experimental import pallas as pl

B = 128
S = 256
DIM = 768
EMB = 256
K = 8192
EPS = 1e-12
G = 8
NIMG = 2 * B
KT = 1024


def _np_consts():
    sel1 = np.zeros((9 * 64, 256), np.float32)
    for dh in range(3):
        for dw in range(3):
            t = dh * 3 + dw
            for i in range(8):
                for j in range(8):
                    h, w = 2 * i + dh - 1, 2 * j + dw - 1
                    if 0 <= h < 16 and 0 <= w < 16:
                        sel1[t * 64 + i * 8 + j, h * 16 + w] = 1.0
    sel2 = np.zeros((9 * 16 * G, 64 * G), np.float32)
    for dh in range(3):
        for dw in range(3):
            t = dh * 3 + dw
            for g in range(G):
                for i in range(4):
                    for j in range(4):
                        h, w = 2 * i + dh - 1, 2 * j + dw - 1
                        if 0 <= h < 8 and 0 <= w < 8:
                            sel2[t * 16 * G + g * 16 + i * 4 + j,
                                 g * 64 + h * 8 + w] = 1.0
    sel3 = np.zeros((9 * 4 * G, 16 * G), np.float32)
    for dh in range(3):
        for dw in range(3):
            t = dh * 3 + dw
            for g in range(G):
                for i in range(2):
                    for j in range(2):
                        h, w = i + dh, j + dw
                        sel3[t * 4 * G + g * 4 + i * 2 + j,
                             g * 16 + h * 4 + w] = 1.0
    gcat = np.zeros((4 * EMB, 4 * EMB), np.float32)
    for i in range(4):
        for j in range(EMB):
            p, c = j // 64, 4 * (j % 64) + i
            gcat[p * EMB + c, i * EMB + j] = 1.0
    return sel1, sel2, sel3, gcat


_SEL1, _SEL2, _SEL3, _GCAT = _np_consts()


def _encoder_body(x_ref, win_ref, bin_ref, sel1_ref, sel2_ref, sel3_ref,
                  w1_ref, b1_ref, w2_ref, b2_ref, w3_ref, b3_ref, z_ref):
    f32 = jnp.float32
    X = x_ref[...].reshape(G * S, DIM)
    Y = jnp.dot(X, win_ref[...], preferred_element_type=f32) + bin_ref[...]
    sel1 = sel1_ref[...]
    t1 = jnp.concatenate(
        [jnp.dot(sel1, Y[g * S:(g + 1) * S], preferred_element_type=f32)[None]
         for g in range(G)], axis=0)
    a1 = jnp.zeros((G * 64, EMB), f32)
    for t in range(9):
        xt = t1[:, t * 64:(t + 1) * 64, :].reshape(G * 64, EMB)
        a1 = a1 + jnp.dot(xt, w1_ref[t], preferred_element_type=f32)
    a1 = jax.nn.relu(a1 + b1_ref[...])
    t2 = jnp.dot(sel2_ref[...], a1, preferred_element_type=f32)
    a2 = jnp.zeros((G * 16, EMB), f32)
    for t in range(9):
        a2 = a2 + jnp.dot(t2[t * 16 * G:(t + 1) * 16 * G], w2_ref[t],
                          preferred_element_type=f32)
    a2 = jax.nn.relu(a2 + b2_ref[...])
    t3 = jnp.dot(sel3_ref[...], a2, preferred_element_type=f32)
    z = jnp.zeros((G * 4, EMB), f32)
    for t in range(9):
        z = z + jnp.dot(t3[t * 4 * G:(t + 1) * 4 * G], w3_ref[t],
                        preferred_element_type=f32)
    z_ref[...] = z + b3_ref[...]


def _vq_body(zf_ref, zl_ref, cbt_ref, cb_ref, noise_ref, q_ref, idx_ref):
    f32 = jnp.float32
    x = zl_ref[...] - zf_ref[...]
    xn2 = jnp.sum(x * x, axis=1, keepdims=True)
    best = jnp.full((4 * B, 1), jnp.inf, f32)
    bidx = jnp.zeros((4 * B, 1), jnp.int32)
    lane = jax.lax.broadcasted_iota(jnp.int32, (4 * B, KT), 1)
    for t in range(K // KT):
        cbt = cbt_ref[:, t * KT:(t + 1) * KT]
        cn2 = jnp.sum(cbt * cbt, axis=0, keepdims=True)
        s = xn2 - 2.0 * jnp.dot(x, cbt, preferred_element_type=f32) + cn2
        m = jnp.min(s, axis=1, keepdims=True)
        li = jnp.min(jnp.where(s <= m, lane, K), axis=1, keepdims=True) + t * KT
        upd = m < best
        bidx = jnp.where(upd, li, bidx)
        best = jnp.where(upd, m, best)
    hq = jnp.zeros((4 * B, EMB), f32)
    for t in range(K // KT):
        oh = (bidx == (lane + t * KT)).astype(f32)
        hq = hq + jnp.dot(oh, cb_ref[t * KT:(t + 1) * KT, :],
                          preferred_element_type=f32)
    r = x - hq
    nres = jnp.sqrt(jnp.sum(r * r, axis=1, keepdims=True))
    noise = noise_ref[...]
    nrand = jnp.sqrt(jnp.sum(noise * noise, axis=1, keepdims=True))
    q_ref[...] = x + (nres / nrand + EPS) * noise
    idx_ref[...] = bidx


def _out_body(qf_ref, idx_ref, gcat_ref, wout_ref, bout_ref, out_ref, ppl_ref):
    f32 = jnp.float32
    qd = jnp.dot(qf_ref[...], gcat_ref[...], preferred_element_type=f32)
    outs = []
    for i in range(4):
        o = jnp.dot(qd[:, i * EMB:(i + 1) * EMB], wout_ref[...],
                    preferred_element_type=f32) + bout_ref[...]
        outs.append(o[None])
    out_ref[...] = jnp.concatenate(outs, axis=0)
    idx = idx_ref[...]
    lane = jax.lax.broadcasted_iota(jnp.int32, (4 * B, KT), 1)
    tot = jnp.float32(0.0)
    for t in range(K // KT):
        cnt = jnp.sum((idx == (lane + t * KT)).astype(f32), axis=0,
                      keepdims=True)
        p = cnt / jnp.float32(4 * B)
        tot = tot + jnp.sum(p * jnp.log(p + 1e-10))
    ppl_ref[...] = jnp.full((1, 1), 1.0, f32) * jnp.exp(-tot)


def kernel(input_data_first, input_data_last, codebooks, W_in, b_in,
           conv1_w, conv1_b, conv2_w, conv2_b, conv3_w, conv3_b,
           W_out, b_out, noise):
    f32 = jnp.float32
    sel1 = jnp.asarray(_SEL1)
    sel2 = jnp.asarray(_SEL2)
    sel3 = jnp.asarray(_SEL3)
    gcat = jnp.asarray(_GCAT)

    w1 = conv1_w.reshape(9, EMB, EMB)
    w2 = conv2_w.reshape(9, EMB, EMB)
    w3 = conv3_w.reshape(9, EMB, EMB)

    full = lambda shape: pl.BlockSpec(shape, lambda g: (0,) * len(shape))
    encode = pl.pallas_call(
        _encoder_body,
        grid=(B // G,),
        in_specs=[
            pl.BlockSpec((G, S, DIM), lambda g: (g, 0, 0)),
            full((DIM, EMB)),
            full((1, EMB)),
            full((9 * 64, 256)),
            full((9 * 16 * G, 64 * G)),
            full((9 * 4 * G, 16 * G)),
            full((9, EMB, EMB)),
            full((1, EMB)),
            full((9, EMB, EMB)),
            full((1, EMB)),
            full((9, EMB, EMB)),
            full((1, EMB)),
        ],
        out_specs=pl.BlockSpec((G * 4, EMB), lambda g: (g, 0)),
        out_shape=jax.ShapeDtypeStruct((B * 4, EMB), f32),
    )
    wargs = (W_in, b_in.reshape(1, EMB), sel1, sel2, sel3,
             w1, conv1_b.reshape(1, EMB), w2, conv2_b.reshape(1, EMB),
             w3, conv3_b.reshape(1, EMB))
    zf = encode(input_data_first, *wargs)
    zl = encode(input_data_last, *wargs)

    q, idx = pl.pallas_call(
        _vq_body,
        out_shape=(jax.ShapeDtypeStruct((4 * B, EMB), f32),
                   jax.ShapeDtypeStruct((4 * B, 1), jnp.int32)),
    )(zf, zl, codebooks.T, codebooks, noise)

    qf = q.reshape(B, 4 * EMB)
    out4, ppl = pl.pallas_call(
        _out_body,
        out_shape=(jax.ShapeDtypeStruct((4, B, DIM), f32),
                   jax.ShapeDtypeStruct((1, 1), f32)),
    )(qf, idx, gcat, W_out, b_out.reshape(1, DIM))

    out = jnp.transpose(out4, (1, 0, 2))
    return out, ppl.reshape(()), idx.reshape(4 * B)


# speedup vs baseline: 1.5284x; 1.0007x over previous
import numpy as np
import jax
import jax.numpy as jnp
from jax.experimental import pallas as pl

B = 128
S = 256
DIM = 768
EMB = 256
K = 8192
EPS = 1e-12
G = 8
NIMG = 2 * B
KT = 1024


def _np_consts():
    sel1 = np.zeros((9 * 64, 256), np.float32)
    for dh in range(3):
        for dw in range(3):
            t = dh * 3 + dw
            for i in range(8):
                for j in range(8):
                    h, w = 2 * i + dh - 1, 2 * j + dw - 1
                    if 0 <= h < 16 and 0 <= w < 16:
                        sel1[t * 64 + i * 8 + j, h * 16 + w] = 1.0
    sel2 = np.zeros((9 * 16 * G, 64 * G), np.float32)
    for dh in range(3):
        for dw in range(3):
            t = dh * 3 + dw
            for g in range(G):
                for i in range(4):
                    for j in range(4):
                        h, w = 2 * i + dh - 1, 2 * j + dw - 1
                        if 0 <= h < 8 and 0 <= w < 8:
                            sel2[t * 16 * G + g * 16 + i * 4 + j,
                                 g * 64 + h * 8 + w] = 1.0
    sel3 = np.zeros((9 * 4 * G, 16 * G), np.float32)
    for dh in range(3):
        for dw in range(3):
            t = dh * 3 + dw
            for g in range(G):
                for i in range(2):
                    for j in range(2):
                        h, w = i + dh, j + dw
                        sel3[t * 4 * G + g * 4 + i * 2 + j,
                             g * 16 + h * 4 + w] = 1.0
    gcat = np.zeros((4 * EMB, 4 * EMB), np.float32)
    for i in range(4):
        for j in range(EMB):
            p, c = j // 64, 4 * (j % 64) + i
            gcat[p * EMB + c, i * EMB + j] = 1.0
    return sel1, sel2, sel3, gcat


_SEL1, _SEL2, _SEL3, _GCAT = _np_consts()


def _encoder_body(x_ref, win_ref, bin_ref, sel1_ref, sel2_ref, sel3_ref,
                  w1_ref, b1_ref, w2_ref, b2_ref, w3_ref, b3_ref, z_ref):
    f32 = jnp.float32
    X = x_ref[...].reshape(G * S, DIM)
    Y = jnp.dot(X, win_ref[...], preferred_element_type=f32) + bin_ref[...]
    sel1 = sel1_ref[...]
    p1 = jnp.concatenate(
        [jnp.concatenate(
            [t1g[t * 64:(t + 1) * 64] for t in range(9)], axis=1)
         for g in range(G)
         for t1g in [jnp.dot(sel1, Y[g * S:(g + 1) * S],
                             preferred_element_type=f32)]], axis=0)
    a1 = jnp.dot(p1, w1_ref[...], preferred_element_type=f32)
    a1 = jax.nn.relu(a1 + b1_ref[...])
    t2 = jnp.dot(sel2_ref[...], a1, preferred_element_type=f32)
    p2 = jnp.concatenate([t2[t * 16 * G:(t + 1) * 16 * G] for t in range(9)],
                         axis=1)
    a2 = jax.nn.relu(jnp.dot(p2, w2_ref[...], preferred_element_type=f32)
                     + b2_ref[...])
    t3 = jnp.dot(sel3_ref[...], a2, preferred_element_type=f32)
    p3 = jnp.concatenate([t3[t * 4 * G:(t + 1) * 4 * G] for t in range(9)],
                         axis=1)
    z = jnp.dot(p3, w3_ref[...], preferred_element_type=f32)
    z_ref[...] = z + b3_ref[...]


def _vq_body(zf_ref, zl_ref, cbt_ref, cb_ref, noise_ref, q_ref, idx_ref):
    f32 = jnp.float32
    x = zl_ref[...] - zf_ref[...]
    xn2 = jnp.sum(x * x, axis=1, keepdims=True)
    best = jnp.full((4 * B, 1), jnp.inf, f32)
    bidx = jnp.zeros((4 * B, 1), jnp.int32)
    lane = jax.lax.broadcasted_iota(jnp.int32, (4 * B, KT), 1)
    for t in range(K // KT):
        cbt = cbt_ref[:, t * KT:(t + 1) * KT]
        cn2 = jnp.sum(cbt * cbt, axis=0, keepdims=True)
        s = xn2 - 2.0 * jnp.dot(x, cbt, preferred_element_type=f32) + cn2
        m = jnp.min(s, axis=1, keepdims=True)
        li = jnp.min(jnp.where(s <= m, lane, K), axis=1, keepdims=True) + t * KT
        upd = m < best
        bidx = jnp.where(upd, li, bidx)
        best = jnp.where(upd, m, best)
    hq = jnp.zeros((4 * B, EMB), f32)
    for t in range(K // KT):
        oh = (bidx == (lane + t * KT)).astype(f32)
        hq = hq + jnp.dot(oh, cb_ref[t * KT:(t + 1) * KT, :],
                          preferred_element_type=f32)
    r = x - hq
    nres = jnp.sqrt(jnp.sum(r * r, axis=1, keepdims=True))
    noise = noise_ref[...]
    nrand = jnp.sqrt(jnp.sum(noise * noise, axis=1, keepdims=True))
    q_ref[...] = x + (nres / nrand + EPS) * noise
    idx_ref[...] = bidx


def _out_body(qf_ref, idx_ref, gcat_ref, wout_ref, bout_ref, out_ref, ppl_ref):
    f32 = jnp.float32
    qd = jnp.dot(qf_ref[...], gcat_ref[...], preferred_element_type=f32)
    outs = []
    for i in range(4):
        o = jnp.dot(qd[:, i * EMB:(i + 1) * EMB], wout_ref[...],
                    preferred_element_type=f32) + bout_ref[...]
        outs.append(o[None])
    out_ref[...] = jnp.concatenate(outs, axis=0)
    idx = idx_ref[...]
    lane = jax.lax.broadcasted_iota(jnp.int32, (4 * B, KT), 1)
    tot = jnp.float32(0.0)
    for t in range(K // KT):
        cnt = jnp.sum((idx == (lane + t * KT)).astype(f32), axis=0,
                      keepdims=True)
        p = cnt / jnp.float32(4 * B)
        tot = tot + jnp.sum(p * jnp.log(p + 1e-10))
    ppl_ref[...] = jnp.full((1, 1), 1.0, f32) * jnp.exp(-tot)


def kernel(input_data_first, input_data_last, codebooks, W_in, b_in,
           conv1_w, conv1_b, conv2_w, conv2_b, conv3_w, conv3_b,
           W_out, b_out, noise):
    f32 = jnp.float32
    sel1 = jnp.asarray(_SEL1)
    sel2 = jnp.asarray(_SEL2)
    sel3 = jnp.asarray(_SEL3)
    gcat = jnp.asarray(_GCAT)

    w1 = conv1_w.reshape(9 * EMB, EMB)
    w2 = conv2_w.reshape(9 * EMB, EMB)
    w3 = conv3_w.reshape(9 * EMB, EMB)

    full = lambda shape: pl.BlockSpec(shape, lambda g: (0,) * len(shape))
    encode = pl.pallas_call(
        _encoder_body,
        grid=(B // G,),
        in_specs=[
            pl.BlockSpec((G, S, DIM), lambda g: (g, 0, 0)),
            full((DIM, EMB)),
            full((1, EMB)),
            full((9 * 64, 256)),
            full((9 * 16 * G, 64 * G)),
            full((9 * 4 * G, 16 * G)),
            full((9 * EMB, EMB)),
            full((1, EMB)),
            full((9 * EMB, EMB)),
            full((1, EMB)),
            full((9 * EMB, EMB)),
            full((1, EMB)),
        ],
        out_specs=pl.BlockSpec((G * 4, EMB), lambda g: (g, 0)),
        out_shape=jax.ShapeDtypeStruct((B * 4, EMB), f32),
    )
    wargs = (W_in, b_in.reshape(1, EMB), sel1, sel2, sel3,
             w1, conv1_b.reshape(1, EMB), w2, conv2_b.reshape(1, EMB),
             w3, conv3_b.reshape(1, EMB))
    zf = encode(input_data_first, *wargs)
    zl = encode(input_data_last, *wargs)

    q, idx = pl.pallas_call(
        _vq_body,
        out_shape=(jax.ShapeDtypeStruct((4 * B, EMB), f32),
                   jax.ShapeDtypeStruct((4 * B, 1), jnp.int32)),
    )(zf, zl, codebooks.T, codebooks, noise)

    qf = q.reshape(B, 4 * EMB)
    out4, ppl = pl.pallas_call(
        _out_body,
        out_shape=(jax.ShapeDtypeStruct((4, B, DIM), f32),
                   jax.ShapeDtypeStruct((1, 1), f32)),
    )(qf, idx, gcat, W_out, b_out.reshape(1, DIM))

    out = jnp.transpose(out4, (1, 0, 2))
    return out, ppl.reshape(()), idx.reshape(4 * B)
